# Initial kernel scaffold; baseline (speedup 1.0000x reference)
import jax
import jax.numpy as jnp
from jax.experimental import pallas as pl


def kernel(x, edge_index, batch, lw, lin0_W, lin0_b, bn0_g, bn0_b, lin1_W, lin1_b, bn1_g, bn1_b, lin2_W, lin2_b, bn2_g, bn2_b, conv0_W, conv0_b, cbn0_g, cbn0_b, conv1_W, conv1_b, cbn1_g, cbn1_b):
    raise NotImplementedError("write your pallas kernel here")


# speedup vs baseline: 3.6040x; 3.6040x over previous
import functools

import jax
import jax.numpy as jnp
from jax import lax
from jax.experimental import pallas as pl
from jax.experimental.pallas import tpu as pltpu
from jax.experimental.pallas import tpu_sc as plsc

_N = 10000
_E = 320000
_G = 32
_D = 128

_NCORES = 2
_NSUB = 16
_DH = _D // 2
_EPT = _E // _NSUB
_BATCH = 128
_NB = _EPT // _BATCH
_TAIL = _EPT - _NB * _BATCH
_RPT = 632
_RLAST = _N - (_NSUB - 1) * _RPT


def _make_seg_sum(n, d):
    mesh = plsc.VectorSubcoreMesh(core_axis_name="c", subcore_axis_name="s")
    scratch = [
        pltpu.VMEM((_BATCH,), jnp.int32),
        pltpu.VMEM((_BATCH,), jnp.int32),
        pltpu.VMEM((_BATCH, d), jnp.float32),
        pltpu.VMEM_SHARED((n, d), jnp.float32),
        pltpu.VMEM((_RPT, d), jnp.float32),
        pltpu.SemaphoreType.DMA,
        pltpu.VMEM((_TAIL,), jnp.int32),
        pltpu.VMEM((_TAIL,), jnp.int32),
        pltpu.VMEM((_TAIL, d), jnp.float32),
    ]

    @functools.partial(
        pl.kernel,
        out_type=jax.ShapeDtypeStruct((2 * n, d), jnp.float32),
        mesh=mesh,
        scratch_types=scratch,
        compiler_params=pltpu.CompilerParams(use_tc_tiling_on_sc=False),
    )
    def seg(table, srcs, dsts, zeros, out,
            src_v, dst_v, rows_v, acc, bounce, sem, tsrc_v, tdst_v, trows_v):
        c = lax.axis_index("c")
        s = lax.axis_index("s")
        r0 = s * _RPT
        coff = c * n
        pltpu.sync_copy(zeros, bounce)

        @pl.when(s < _NSUB - 1)
        def _():
            pltpu.sync_copy(bounce.at[pl.ds(0, _RPT)], acc.at[pl.ds(r0, _RPT)])

        @pl.when(s == _NSUB - 1)
        def _():
            pltpu.sync_copy(bounce.at[pl.ds(0, _RLAST)], acc.at[pl.ds(r0, _RLAST)])

        plsc.subcore_barrier()

        ebase = s * _EPT

        def offset_src(ref, nvec):
            for j in range(nvec):
                sl = pl.ds(j * 16, 16)
                ref[sl] = ref[sl] + coff

        def body(i, carry):
            b = ebase + i * _BATCH
            pltpu.sync_copy(srcs.at[pl.ds(b, _BATCH)], src_v)
            pltpu.sync_copy(dsts.at[pl.ds(b, _BATCH)], dst_v)
            offset_src(src_v, _BATCH // 16)
            pltpu.async_copy(table.at[src_v], rows_v, sem).wait()
            pltpu.sync_copy(rows_v, acc.at[dst_v], add=True)
            return carry

        lax.fori_loop(0, _NB, body, 0)

        b = ebase + _NB * _BATCH
        pltpu.sync_copy(srcs.at[pl.ds(b, _TAIL)], tsrc_v)
        pltpu.sync_copy(dsts.at[pl.ds(b, _TAIL)], tdst_v)
        offset_src(tsrc_v, _TAIL // 16)
        pltpu.async_copy(table.at[tsrc_v], trows_v, sem).wait()
        pltpu.sync_copy(trows_v, acc.at[tdst_v], add=True)

        plsc.subcore_barrier()

        @pl.when(s < _NSUB - 1)
        def _():
            pltpu.sync_copy(acc.at[pl.ds(r0, _RPT)], bounce.at[pl.ds(0, _RPT)])
            pltpu.sync_copy(bounce.at[pl.ds(0, _RPT)],
                            out.at[pl.ds(c * n + r0, _RPT)])

        @pl.when(s == _NSUB - 1)
        def _():
            pltpu.sync_copy(acc.at[pl.ds(r0, _RLAST)], bounce.at[pl.ds(0, _RLAST)])
            pltpu.sync_copy(bounce.at[pl.ds(0, _RLAST)],
                            out.at[pl.ds(c * n + r0, _RLAST)])

    return seg


_SEG_SUM_CACHE = []


def _seg_sum(table_split, src, dst, zeros):
    if not _SEG_SUM_CACHE:
        _SEG_SUM_CACHE.append(_make_seg_sum(_N, _DH))
    return _SEG_SUM_CACHE[0](table_split, src, dst, zeros)


def _split(a):
    return jnp.concatenate([a[:, :_DH], a[:, _DH:]], axis=0)


def _unsplit(p):
    return jnp.concatenate([p[0:_N, :], p[_N:2 * _N, :]], axis=1)


_BR = 1000
_NBLK = _N // _BR


def _elu(y):
    return jnp.where(y > 0, y, jnp.exp(jnp.minimum(y, 0.0)) - 1.0)


def _lbe_body(has_agg, f):
    def body(*refs):
        if has_agg:
            (x_ref, pa_ref, pb_ref, w_ref, b_ref, g_ref, bb_ref,
             z_ref, y_acc, s1, s2) = refs
        else:
            (x_ref, w_ref, b_ref, g_ref, bb_ref, z_ref, y_acc, s1, s2) = refs
        p = pl.program_id(0)
        i = pl.program_id(1)

        @pl.when(p == 0)
        def _():
            t = x_ref[...]
            if has_agg:
                t = t + jnp.concatenate([pa_ref[...], pb_ref[...]], axis=1)
            y = jnp.dot(t, w_ref[...], preferred_element_type=jnp.float32) \
                + b_ref[...]
            y_acc[pl.ds(i * _BR, _BR), :] = y
            i1 = jnp.sum(y, axis=0, keepdims=True)
            i2 = jnp.sum(y * y, axis=0, keepdims=True)
            s1[0:1, :] = jnp.where(i == 0, i1, s1[0:1, :] + i1)
            s2[0:1, :] = jnp.where(i == 0, i2, s2[0:1, :] + i2)

        @pl.when(p == 1)
        def _():
            y = y_acc[pl.ds(i * _BR, _BR), :]
            m = s1[0:1, :] * (1.0 / _N)
            v = s2[0:1, :] * (1.0 / _N) - m * m
            z_ref[...] = _elu((y - m) * lax.rsqrt(v + 1e-5) * g_ref[...]
                              + bb_ref[...])

    return body


def _lbe(x, w, b, g, bb, p=None):
    k = x.shape[1]
    f = w.shape[1]
    has_agg = p is not None
    row = lambda pp, ii: (ii, 0)
    in_specs = [pl.BlockSpec((_BR, k), row)]
    args = [x]
    if has_agg:
        in_specs += [pl.BlockSpec((_BR, _DH), row),
                     pl.BlockSpec((_BR, _DH), lambda pp, ii: (ii + _NBLK, 0))]
        args += [p, p]
    in_specs += [pl.BlockSpec((k, f), lambda pp, ii: (0, 0))] + \
        [pl.BlockSpec((1, f), lambda pp, ii: (0, 0))] * 3
    args += [w, b.reshape(1, -1), g.reshape(1, -1), bb.reshape(1, -1)]
    return pl.pallas_call(
        _lbe_body(has_agg, f),
        grid=(2, _NBLK),
        in_specs=in_specs,
        out_specs=pl.BlockSpec((_BR, f), row),
        out_shape=jax.ShapeDtypeStruct((_N, f), jnp.float32),
        scratch_shapes=[pltpu.VMEM((_N, f), jnp.float32),
                        pltpu.VMEM((8, f), jnp.float32),
                        pltpu.VMEM((8, f), jnp.float32)],
    )(*args)


def _tc_pool(z0_ref, z1_ref, z2_ref, lws_ref, batch_ref, out_ref, zz_ref, acc):
    i = pl.program_id(0)
    lws = lws_ref[...]
    z0 = z0_ref[...] * lws[0, 0]
    z1 = z1_ref[...] * lws[0, 1]
    z2 = z2_ref[...] * lws[0, 2]
    zz_ref[...] = z0 + z1 + z2
    big = jnp.concatenate([z0, z1, z2], axis=1)
    batch = batch_ref[...]
    ninf = jnp.float32(-jnp.inf)
    rows = []
    for g in range(_G):
        rows.append(jnp.max(jnp.where(batch == g, big, ninf), axis=0))
    blockmax = jnp.stack(rows)
    prev = jnp.where(i == 0, jnp.full((_G, 30), ninf), acc[...])
    acc[...] = jnp.maximum(prev, blockmax)

    @pl.when(i == _NBLK - 1)
    def _():
        oc = acc[...]
        out_ref[...] = oc[:, 0:10] + oc[:, 10:20] + oc[:, 20:30]


def _f32(*shape):
    return jax.ShapeDtypeStruct(shape, jnp.float32)


def kernel(x, edge_index, batch, lw,
           lin0_W, lin0_b, bn0_g, bn0_b,
           lin1_W, lin1_b, bn1_g, bn1_b,
           lin2_W, lin2_b, bn2_g, bn2_b,
           conv0_W, conv0_b, cbn0_g, cbn0_b,
           conv1_W, conv1_b, cbn1_g, cbn1_b):
    src = edge_index[0]
    dst = edge_index[1]
    zeros = jnp.zeros((_RPT, _DH), jnp.float32)

    p1 = _seg_sum(_split(x), src, dst, zeros)

    z0 = _lbe(x, lin0_W, lin0_b, bn0_g, bn0_b)

    h1 = _lbe(x, conv0_W, conv0_b, cbn0_g, cbn0_b, p=p1)
    z1 = _lbe(h1, lin1_W, lin1_b, bn1_g, bn1_b)

    p2 = _seg_sum(_split(h1), src, dst, zeros)

    h2 = _lbe(h1, conv1_W, conv1_b, cbn1_g, cbn1_b, p=p2)
    z2 = _lbe(h2, lin2_W, lin2_b, bn2_g, bn2_b)

    rowspec = pl.BlockSpec((_BR, 10), lambda i: (i, 0))
    whole = lambda shp: pl.BlockSpec(shp, lambda i: (0, 0))
    out, zz = pl.pallas_call(
        _tc_pool,
        grid=(_NBLK,),
        in_specs=[rowspec, rowspec, rowspec, whole((1, 3)),
                  pl.BlockSpec((_BR, 1), lambda i: (i, 0))],
        out_specs=[whole((_G, 10)), rowspec],
        out_shape=[_f32(_G, 10), _f32(_N, 10)],
        scratch_shapes=[pltpu.VMEM((_G, 30), jnp.float32)],
    )(z0, z1, z2, lw.reshape(1, 3), batch.reshape(-1, 1))
    return (out, zz, h2)


# speedup vs baseline: 6.5833x; 1.8267x over previous
import functools

import jax
import jax.numpy as jnp
from jax import lax
from jax.experimental import pallas as pl
from jax.experimental.pallas import tpu as pltpu
from jax.experimental.pallas import tpu_sc as plsc

_N = 10000
_E = 320000
_G = 32
_D = 128

_NCORES = 2
_NSUB = 16
_DH = _D // 2
_EPT = _E // _NSUB
_BATCH = 128
_NBAT = (_EPT + _BATCH - 1) // _BATCH
_PAIRS = (_NBAT - 1) // 2
_NPAD = _N + 8
_RPT = 632
_RLAST = _N - (_NSUB - 1) * _RPT
_ZLAST = _NPAD - (_NSUB - 1) * _RPT


def _make_seg_sum(n, d):
    mesh = plsc.VectorSubcoreMesh(core_axis_name="c", subcore_axis_name="s")
    scratch = [
        pltpu.VMEM((_NBAT, _BATCH), jnp.int32),
        pltpu.VMEM((_NBAT, _BATCH), jnp.int32),
        pltpu.VMEM((_BATCH, d), jnp.float32),
        pltpu.VMEM((_BATCH, d), jnp.float32),
        pltpu.VMEM_SHARED((_NPAD, d), jnp.float32),
        pltpu.SemaphoreType.DMA,
        pltpu.SemaphoreType.DMA,
    ]

    @functools.partial(
        pl.kernel,
        out_type=jax.ShapeDtypeStruct((2 * n, d), jnp.float32),
        mesh=mesh,
        scratch_types=scratch,
        compiler_params=pltpu.CompilerParams(use_tc_tiling_on_sc=False),
    )
    def seg(table, srcs, dsts, zeros, out,
            src_t, dst_t, rows0, rows1, acc, sem0, sem1):
        c = lax.axis_index("c")
        s = lax.axis_index("s")
        r0 = s * _RPT
        rows = (rows0, rows1)
        sems = (sem0, sem1)

        pltpu.sync_copy(srcs.at[c * _NSUB + s], src_t)
        pltpu.sync_copy(dsts.at[s], dst_t)

        pltpu.sync_copy(zeros, rows0)
        for k in range(4):
            pltpu.sync_copy(rows0, acc.at[pl.ds(r0 + 128 * k, 128)])

        @pl.when(s < _NSUB - 1)
        def _():
            pltpu.sync_copy(rows0.at[pl.ds(0, 120)],
                            acc.at[pl.ds(r0 + 512, 120)])

        @pl.when(s == _NSUB - 1)
        def _():
            pltpu.sync_copy(rows0.at[pl.ds(0, 16)],
                            acc.at[pl.ds(r0 + 512, 16)])

        plsc.subcore_barrier()

        def gstart(k, slot):
            pltpu.async_copy(table.at[src_t.at[k]], rows[slot], sems[slot])

        def gwait(slot):
            pltpu.make_async_copy(table.at[src_t.at[0]], rows[slot],
                                  sems[slot]).wait()

        def scat(k, slot):
            pltpu.sync_copy(rows[slot], acc.at[dst_t.at[k]], add=True)

        gstart(0, 0)

        def body(j, carry):
            k = 2 * j
            gstart(k + 1, 1)
            gwait(0)
            scat(k, 0)
            gstart(k + 2, 0)
            gwait(1)
            scat(k + 1, 1)
            return carry

        lax.fori_loop(0, _PAIRS, body, 0)
        gwait(0)
        scat(_NBAT - 1, 0)

        plsc.subcore_barrier()

        def chunk_out(off, size, buf):
            pltpu.sync_copy(acc.at[pl.ds(r0 + off, size)],
                            buf.at[pl.ds(0, size)])
            pltpu.sync_copy(buf.at[pl.ds(0, size)],
                            out.at[pl.ds(c * n + r0 + off, size)])

        for k in range(4):
            chunk_out(128 * k, 128, rows[k % 2])

        @pl.when(s < _NSUB - 1)
        def _():
            chunk_out(512, 120, rows0)

        @pl.when(s == _NSUB - 1)
        def _():
            chunk_out(512, 8, rows0)

    return seg


_SEG_SUM_CACHE = []


def _seg_sum(table_split, srcs, dsts, zeros):
    if not _SEG_SUM_CACHE:
        _SEG_SUM_CACHE.append(_make_seg_sum(_N, _DH))
    return _SEG_SUM_CACHE[0](table_split, srcs, dsts, zeros)


def _prep_edges(src, dst):
    pad = _NSUB * _NBAT * _BATCH - _E
    s2 = jnp.pad(src.reshape(_NSUB, _EPT), ((0, 0), (0, _NBAT * _BATCH - _EPT)))
    srcs = jnp.concatenate([s2, s2 + _N], axis=0).reshape(
        2 * _NSUB, _NBAT, _BATCH)
    d2 = jnp.pad(dst.reshape(_NSUB, _EPT),
                 ((0, 0), (0, _NBAT * _BATCH - _EPT)),
                 constant_values=_N)
    dsts = d2.reshape(_NSUB, _NBAT, _BATCH)
    return srcs, dsts


def _split(a):
    return jnp.concatenate([a[:, :_DH], a[:, _DH:]], axis=0)


def _unsplit(p):
    return jnp.concatenate([p[0:_N, :], p[_N:2 * _N, :]], axis=1)


_BR = 1000
_NBLK = _N // _BR


def _elu(y):
    return jnp.where(y > 0, y, jnp.exp(jnp.minimum(y, 0.0)) - 1.0)


def _lbe_body(has_agg, f):
    def body(*refs):
        if has_agg:
            (x_ref, pa_ref, pb_ref, w_ref, b_ref, g_ref, bb_ref,
             z_ref, y_acc, s1, s2) = refs
        else:
            (x_ref, w_ref, b_ref, g_ref, bb_ref, z_ref, y_acc, s1, s2) = refs
        p = pl.program_id(0)
        i = pl.program_id(1)

        @pl.when(p == 0)
        def _():
            t = x_ref[...]
            if has_agg:
                t = t + jnp.concatenate([pa_ref[...], pb_ref[...]], axis=1)
            y = jnp.dot(t, w_ref[...], preferred_element_type=jnp.float32) \
                + b_ref[...]
            y_acc[pl.ds(i * _BR, _BR), :] = y
            i1 = jnp.sum(y, axis=0, keepdims=True)
            i2 = jnp.sum(y * y, axis=0, keepdims=True)
            s1[0:1, :] = jnp.where(i == 0, i1, s1[0:1, :] + i1)
            s2[0:1, :] = jnp.where(i == 0, i2, s2[0:1, :] + i2)

        @pl.when(p == 1)
        def _():
            y = y_acc[pl.ds(i * _BR, _BR), :]
            m = s1[0:1, :] * (1.0 / _N)
            v = s2[0:1, :] * (1.0 / _N) - m * m
            z_ref[...] = _elu((y - m) * lax.rsqrt(v + 1e-5) * g_ref[...]
                              + bb_ref[...])

    return body


def _lbe(x, w, b, g, bb, p=None):
    k = x.shape[1]
    f = w.shape[1]
    has_agg = p is not None
    row = lambda pp, ii: (ii, 0)
    in_specs = [pl.BlockSpec((_BR, k), row)]
    args = [x]
    if has_agg:
        in_specs += [pl.BlockSpec((_BR, _DH), row),
                     pl.BlockSpec((_BR, _DH), lambda pp, ii: (ii + _NBLK, 0))]
        args += [p, p]
    in_specs += [pl.BlockSpec((k, f), lambda pp, ii: (0, 0))] + \
        [pl.BlockSpec((1, f), lambda pp, ii: (0, 0))] * 3
    args += [w, b.reshape(1, -1), g.reshape(1, -1), bb.reshape(1, -1)]
    return pl.pallas_call(
        _lbe_body(has_agg, f),
        grid=(2, _NBLK),
        in_specs=in_specs,
        out_specs=pl.BlockSpec((_BR, f), row),
        out_shape=jax.ShapeDtypeStruct((_N, f), jnp.float32),
        scratch_shapes=[pltpu.VMEM((_N, f), jnp.float32),
                        pltpu.VMEM((8, f), jnp.float32),
                        pltpu.VMEM((8, f), jnp.float32)],
    )(*args)


def _tc_pool(z0_ref, z1_ref, z2_ref, lws_ref, batch_ref, out_ref, zz_ref, acc):
    i = pl.program_id(0)
    lws = lws_ref[...]
    z0 = z0_ref[...] * lws[0, 0]
    z1 = z1_ref[...] * lws[0, 1]
    z2 = z2_ref[...] * lws[0, 2]
    zz_ref[...] = z0 + z1 + z2
    big = jnp.concatenate([z0, z1, z2], axis=1)
    batch = batch_ref[...]
    ninf = jnp.float32(-jnp.inf)
    rows = []
    for g in range(_G):
        rows.append(jnp.max(jnp.where(batch == g, big, ninf), axis=0))
    blockmax = jnp.stack(rows)
    prev = jnp.where(i == 0, jnp.full((_G, 30), ninf), acc[...])
    acc[...] = jnp.maximum(prev, blockmax)

    @pl.when(i == _NBLK - 1)
    def _():
        oc = acc[...]
        out_ref[...] = oc[:, 0:10] + oc[:, 10:20] + oc[:, 20:30]


def _f32(*shape):
    return jax.ShapeDtypeStruct(shape, jnp.float32)


def kernel(x, edge_index, batch, lw,
           lin0_W, lin0_b, bn0_g, bn0_b,
           lin1_W, lin1_b, bn1_g, bn1_b,
           lin2_W, lin2_b, bn2_g, bn2_b,
           conv0_W, conv0_b, cbn0_g, cbn0_b,
           conv1_W, conv1_b, cbn1_g, cbn1_b):
    srcs, dsts = _prep_edges(edge_index[0], edge_index[1])
    zeros = jnp.zeros((_BATCH, _DH), jnp.float32)

    p1 = _seg_sum(_split(x), srcs, dsts, zeros)

    z0 = _lbe(x, lin0_W, lin0_b, bn0_g, bn0_b)

    h1 = _lbe(x, conv0_W, conv0_b, cbn0_g, cbn0_b, p=p1)
    z1 = _lbe(h1, lin1_W, lin1_b, bn1_g, bn1_b)

    p2 = _seg_sum(_split(h1), srcs, dsts, zeros)

    h2 = _lbe(h1, conv1_W, conv1_b, cbn1_g, cbn1_b, p=p2)
    z2 = _lbe(h2, lin2_W, lin2_b, bn2_g, bn2_b)

    rowspec = pl.BlockSpec((_BR, 10), lambda i: (i, 0))
    whole = lambda shp: pl.BlockSpec(shp, lambda i: (0, 0))
    out, zz = pl.pallas_call(
        _tc_pool,
        grid=(_NBLK,),
        in_specs=[rowspec, rowspec, rowspec, whole((1, 3)),
                  pl.BlockSpec((_BR, 1), lambda i: (i, 0))],
        out_specs=[whole((_G, 10)), rowspec],
        out_shape=[_f32(_G, 10), _f32(_N, 10)],
        scratch_shapes=[pltpu.VMEM((_G, 30), jnp.float32)],
    )(z0, z1, z2, lw.reshape(1, 3), batch.reshape(-1, 1))
    return (out, zz, h2)
